# Initial kernel scaffold; baseline (speedup 1.0000x reference)
import jax
import jax.numpy as jnp
from jax.experimental import pallas as pl


def kernel(x, node_features, W1, b1, W2, b2, W4, b4, W5, b5, W6, b6):
    raise NotImplementedError("write your pallas kernel here")


# speedup vs baseline: 73.3722x; 73.3722x over previous
import jax
import jax.numpy as jnp
from jax.experimental import pallas as pl

_N = 1024
_F = 64
_H = 128
_BI = 8


def _node_kernel(nf, W1, b1, W2, b2, W4, b4, h_out, hT_out):
    h0 = nf[...]
    t = jnp.dot(h0, W1[...], preferred_element_type=jnp.float32) + b1[...]
    t = jnp.where(t >= 0.0, t, 0.01 * t)
    t = jnp.dot(t, W2[...], preferred_element_type=jnp.float32) + b2[...]
    t = jnp.where(t >= 0.0, t, 0.01 * t)
    t = jnp.dot(t, W4[...], preferred_element_type=jnp.float32) + b4[...]
    h = t + h0
    h_out[...] = h
    hT_out[...] = h.T


def _edge_kernel(hi, hT, W5Tt, b5c, w6c, b6d, out0, out1):
    g = pl.program_id(0)
    hrep = jax.lax.broadcast_in_dim(hi[...], (_BI, _H, _F), (0, 2))
    hrep = hrep.reshape(_BI * _H, _F)
    A = hrep * W5Tt[...]
    T = jnp.dot(A, hT[...], preferred_element_type=jnp.float32)
    T = jnp.maximum(T + b5c[...], 0.0)
    U = T * w6c[...]
    D = jnp.sum(U.reshape(_BI, _H, _N), axis=1) + b6d[0, 0]
    p1 = jax.nn.sigmoid(D)
    p0 = 1.0 - p1
    row = jax.lax.broadcasted_iota(jnp.int32, (_BI, _N), 0)
    col = jax.lax.broadcasted_iota(jnp.int32, (_BI, _N), 1)
    diag = col == (g * _BI + row)
    out0[...] = jnp.where(diag, 0.0, p0)
    out1[...] = jnp.where(diag, 0.0, p1)


def kernel(x, node_features, W1, b1, W2, b2, W4, b4, W5, b5, W6, b6):
    f32 = jnp.float32
    h, hT = pl.pallas_call(
        _node_kernel,
        out_shape=(
            jax.ShapeDtypeStruct((_N, _F), f32),
            jax.ShapeDtypeStruct((_F, _N), f32),
        ),
    )(
        node_features,
        W1,
        b1.reshape(1, _H),
        W2,
        b2.reshape(1, _H),
        W4,
        b4.reshape(1, _F),
    )

    W5Tt = jnp.tile(W5.T, (_BI, 1))
    b5c = jnp.tile(b5, _BI).reshape(_BI * _H, 1)
    w6c = jnp.tile(W6[:, 1] - W6[:, 0], _BI).reshape(_BI * _H, 1)
    b6d = (b6[1] - b6[0]).reshape(1, 1)

    out0, out1 = pl.pallas_call(
        _edge_kernel,
        grid=(_N // _BI,),
        in_specs=[
            pl.BlockSpec((_BI, _F), lambda g: (g, 0)),
            pl.BlockSpec((_F, _N), lambda g: (0, 0)),
            pl.BlockSpec((_BI * _H, _F), lambda g: (0, 0)),
            pl.BlockSpec((_BI * _H, 1), lambda g: (0, 0)),
            pl.BlockSpec((_BI * _H, 1), lambda g: (0, 0)),
            pl.BlockSpec((1, 1), lambda g: (0, 0)),
        ],
        out_specs=[
            pl.BlockSpec((_BI, _N), lambda g: (g, 0)),
            pl.BlockSpec((_BI, _N), lambda g: (g, 0)),
        ],
        out_shape=[
            jax.ShapeDtypeStruct((_N, _N), f32),
            jax.ShapeDtypeStruct((_N, _N), f32),
        ],
    )(h, hT, W5Tt, b5c, w6c, b6d)
    return jnp.stack([out0, out1], axis=-1)
